# baseline (speedup 1.0000x reference)
import functools

import jax
import jax.numpy as jnp
from jax import lax
from jax.experimental import pallas as pl
from jax.experimental.pallas import tpu as pltpu
from jax.experimental.pallas import tpu_sc as plsc

_NC = 2
_NS = 16
_L = 16
_NW = _NC * _NS

_B = 16384
_D = 64
_V = 1000000
_NT = (_V + 127) // 128
_NBIN = 256
_TRASH = 250
_NVEC = _B // _L
_CAP = _B + 2 * _L
_NSLOT = 4
_LEAD = 3
_BLK = 512
_EROWS = _B + _BLK

_mesh = plsc.VectorSubcoreMesh(core_axis_name="c", subcore_axis_name="s")


@functools.partial(
    pl.kernel,
    out_type=(jax.ShapeDtypeStruct((_EROWS, 128), jnp.float32),
              jax.ShapeDtypeStruct((_EROWS, 128), jnp.float32)),
    mesh=_mesh,
    compiler_params=pltpu.CompilerParams(needs_layout_passes=False,
                                         use_tc_tiling_on_sc=True),
    scratch_types=[
        pltpu.VMEM((_B // 128, 128), jnp.int32),
        pltpu.VMEM((_CAP,), jnp.int32),
        pltpu.VMEM((_CAP,), jnp.int32),
        pltpu.VMEM((_NSLOT, _D, 128), jnp.float32),
        pltpu.VMEM((_D, 17), jnp.float32),
        pltpu.VMEM((2, _L, 128), jnp.float32),
        pltpu.VMEM((2, _L), jnp.int32),
        pltpu.SMEM((_NBIN,), jnp.int32),
        pltpu.SMEM((_NBIN + 4,), jnp.int32),
        pltpu.SMEM((_NBIN,), jnp.int32),
        pltpu.SMEM((_NBIN,), jnp.int32),
        pltpu.SemaphoreType.DMA((_NSLOT,)),
        pltpu.SemaphoreType.DMA((2,)),
    ],
)
def _sc_gather_phase(idx0_hbm, idx1_hbm, f0t_hbm, f1t_hbm, e0_hbm, e1_hbm,
                     idx_v, own_p, bin_p, blk, chT, ebuf, epos,
                     cnts, starts, cur, flist, sem, esem):
    wid = lax.axis_index("s") * _NC + lax.axis_index("c")
    lanes = lax.iota(jnp.int32, _L)

    for tbl in range(2):
        src_idx = idx0_hbm if tbl == 0 else idx1_hbm
        ft = f0t_hbm if tbl == 0 else f1t_hbm
        e_hbm = e0_hbm if tbl == 0 else e1_hbm

        pltpu.sync_copy(src_idx, idx_v)

        def zero(i, c):
            cnts[i] = 0
            return c
        lax.fori_loop(0, _NBIN, zero, 0)

        def scan(r, off):
            for c in range(8):
                v = idx_v[r, pl.ds(c * _L, _L)]
                t = v // 128
                own = (t % _NW) == wid
                pos = r * 128 + c * _L + lanes
                packed = ((v >> 12) << 22) | ((v % 128) << 15) | pos
                plsc.store_compressed(own_p.at[pl.ds(off, _L)], packed,
                                      mask=own)
                cnt = plsc.all_reduce_population_count(own)
                off = off + cnt[0]
            return off

        n_own = lax.fori_loop(0, _B // 128, scan, 0)

        p_sent = (_TRASH << 22) | _B
        own_p[pl.ds(n_own, _L)] = jnp.full((_L,), p_sent, jnp.int32)

        n_vec = (n_own + 15) // 16

        def count(jv, c):
            ltv = own_p[pl.ds(jv * _L, _L)] >> 22
            for j in range(_L):
                lt = ltv[j]
                cnts[lt] = cnts[lt] + 1
            return c
        lax.fori_loop(0, n_vec, count, 0)

        starts[0] = 0

        def prefix(i, nf):
            s = starts[i]
            c = cnts[i]
            starts[i + 1] = s + c
            cur[i] = s

            @pl.when(jnp.logical_and(c > 0, i < _TRASH))
            def _():
                flist[nf] = i
            return nf + jnp.where(jnp.logical_and(c > 0, i < _TRASH), 1, 0)

        nf = lax.fori_loop(0, _NBIN, prefix, 0)

        one = lanes == 0

        def scat(jv, c):
            pv = own_p[pl.ds(jv * _L, _L)]
            ltv = pv >> 22
            for j in range(_L):
                lt = ltv[j]
                dst = cur[lt]
                cur[lt] = dst + 1
                plsc.store_scatter(bin_p, [jnp.full((_L,), dst, jnp.int32)],
                                   jnp.full((_L,), pv[j], jnp.int32), mask=one)
            return c
        lax.fori_loop(0, n_vec, scat, 0)

        def fire(fi, slot):
            t = flist[fi] * _NW + wid
            col = pl.multiple_of(t * 128, 128)
            pltpu.async_copy(ft.at[:, pl.ds(col, 128)], blk.at[slot],
                             sem.at[slot])

        for p in range(_LEAD):
            @pl.when(p < nf)
            def _(p=p):
                fire(p, p)

        def per_bin(fi, cc):
            @pl.when(fi + _LEAD < nf)
            def _():
                fire(fi + _LEAD, (fi + _LEAD) % _NSLOT)

            slot = fi % _NSLOT
            pltpu.make_async_copy(ft.at[:, pl.ds(0, 128)], blk.at[slot],
                                  sem.at[slot]).wait()

            b_id = flist[fi]
            s = starts[b_id]
            cnt = cnts[b_id]
            slot_spl = jnp.full((_L,), slot, jnp.int32)

            def chunk(ec, cc2):
                base = s + ec * _L
                rem = cnt - ec * _L
                pv = bin_p[pl.ds(base, _L)]
                valid = lanes < rem
                bvm = jnp.where(valid, pv & 32767, _B)
                cv = (pv >> 15) & 127

                buf = cc2 % 2
                @pl.when(cc2 >= 2)
                def _():
                    pltpu.make_async_copy(
                        ebuf.at[buf], e_hbm.at[epos.at[buf]],
                        esem.at[buf]).wait()

                for d in range(_D):
                    g = plsc.load_gather(
                        blk, [slot_spl, jnp.full((_L,), d, jnp.int32), cv])
                    chT[d, pl.ds(0, _L)] = g
                for j in range(_L):
                    jspl = jnp.full((_L,), j, jnp.int32)
                    for q in range(_D // _L):
                        r = plsc.load_gather(chT, [lanes + q * _L, jspl])
                        ebuf[buf, j, pl.ds(q * _L, _L)] = r
                epos[buf, pl.ds(0, _L)] = bvm
                pltpu.async_copy(ebuf.at[buf], e_hbm.at[epos.at[buf]],
                                 esem.at[buf])
                return cc2 + 1

            n_chunk = (cnt + 15) // 16
            return lax.fori_loop(0, n_chunk, chunk, cc)

        cc_end = lax.fori_loop(0, nf, per_bin, 0)

        @pl.when(cc_end >= 1)
        def _():
            buf = (cc_end - 1) % 2
            pltpu.make_async_copy(ebuf.at[buf], e_hbm.at[epos.at[buf]],
                                  esem.at[buf]).wait()

        @pl.when(cc_end >= 2)
        def _():
            buf = (cc_end - 2) % 2
            pltpu.make_async_copy(ebuf.at[buf], e_hbm.at[epos.at[buf]],
                                  esem.at[buf]).wait()


def _reduce_body(e0_ref, e1_ref, out_ref):
    p = e0_ref[:, :_D] * e1_ref[:, :_D]
    out_ref[...] = jnp.sum(p, axis=1).reshape(out_ref.shape)


def kernel(inputs_0, inputs_1, factor_0, factor_1):
    idx0 = inputs_0.reshape(_B // 128, 128)
    idx1 = inputs_1.reshape(_B // 128, 128)
    e0, e1 = _sc_gather_phase(idx0, idx1, factor_0.T, factor_1.T)
    rows = _B // 4
    out = pl.pallas_call(
        _reduce_body,
        out_shape=jax.ShapeDtypeStruct((_B // _BLK, _BLK), jnp.float32),
        grid=(4,),
        in_specs=[pl.BlockSpec((rows, 128), lambda g: (g, 0)),
                  pl.BlockSpec((rows, 128), lambda g: (g, 0))],
        out_specs=pl.BlockSpec((rows // _BLK, _BLK), lambda g: (g, 0)),
    )(e0, e1)
    return out.reshape(_B)


# speedup vs baseline: 19.5386x; 19.5386x over previous
import functools

import jax
import jax.numpy as jnp
from jax import lax
from jax.experimental import pallas as pl
from jax.experimental.pallas import tpu as pltpu
from jax.experimental.pallas import tpu_sc as plsc

_NC = 2
_NS = 16
_L = 16
_NW = _NC * _NS

_B = 16384
_D = 64
_V = 1000000
_BPW = _B // _NW
_NGRP = _BPW // _L
_NSLOT = 6
_LEAD = 5

_mesh = plsc.VectorSubcoreMesh(core_axis_name="c", subcore_axis_name="s")


@functools.partial(
    pl.kernel,
    out_type=jax.ShapeDtypeStruct((_NW, _BPW), jnp.float32),
    mesh=_mesh,
    compiler_params=pltpu.CompilerParams(needs_layout_passes=False,
                                         use_tc_tiling_on_sc=True),
    scratch_types=[
        pltpu.VMEM((_NGRP, _L), jnp.int32),
        pltpu.VMEM((_NGRP, _L), jnp.int32),
        pltpu.SMEM((_BPW,), jnp.int32),
        pltpu.SMEM((_BPW,), jnp.int32),
        pltpu.VMEM((_NSLOT, _D, 128), jnp.float32),
        pltpu.VMEM((_NSLOT, _D, 128), jnp.float32),
        pltpu.VMEM((_BPW,), jnp.float32),
        pltpu.SemaphoreType.DMA((_NSLOT,)),
        pltpu.SemaphoreType.DMA((_NSLOT,)),
    ],
)
def _sc_dot_gather(idx0_hbm, idx1_hbm, f0t_hbm, f1t_hbm, out_hbm,
                   idx0_v, idx1_v, si0, si1, blk0, blk1, out_v, sem0, sem1):
    wid = lax.axis_index("s") * _NC + lax.axis_index("c")

    pltpu.sync_copy(idx0_hbm.at[wid], idx0_v)
    pltpu.sync_copy(idx1_hbm.at[wid], idx1_v)

    def fill(g, carry):
        v0 = idx0_v[g, :]
        v1 = idx1_v[g, :]
        for j in range(_L):
            si0[g * _L + j] = v0[j]
            si1[g * _L + j] = v1[j]
        return carry

    lax.fori_loop(0, _NGRP, fill, 0)

    def fire(k, slot):
        t0 = pl.multiple_of((si0[k] // 128) * 128, 128)
        t1 = pl.multiple_of((si1[k] // 128) * 128, 128)
        pltpu.async_copy(f0t_hbm.at[:, pl.ds(t0, 128)], blk0.at[slot],
                         sem0.at[slot])
        pltpu.async_copy(f1t_hbm.at[:, pl.ds(t1, 128)], blk1.at[slot],
                         sem1.at[slot])

    for k in range(_LEAD):
        fire(k, k)

    lanes = lax.iota(jnp.int32, _L)
    last_lane = lanes == (_L - 1)

    def body(i, carry):
        @pl.when(i < _BPW - _LEAD)
        def _():
            fire(i + _LEAD, (i + _LEAD) % _NSLOT)

        slot = i % _NSLOT
        pltpu.make_async_copy(f0t_hbm.at[:, pl.ds(0, 128)], blk0.at[slot],
                              sem0.at[slot]).wait()
        pltpu.make_async_copy(f1t_hbm.at[:, pl.ds(0, 128)], blk1.at[slot],
                              sem1.at[slot]).wait()

        c0 = jnp.full((_L,), si0[i] % 128, jnp.int32)
        c1 = jnp.full((_L,), si1[i] % 128, jnp.int32)
        slot_spl = jnp.full((_L,), slot, jnp.int32)
        acc = None
        for q in range(_D // _L):
            rows = lanes + (q * _L)
            e0 = plsc.load_gather(blk0, [slot_spl, rows, c0])
            e1 = plsc.load_gather(blk1, [slot_spl, rows, c1])
            p = e0 * e1
            acc = p if acc is None else acc + p
        plsc.store_scatter(out_v, [jnp.full((_L,), i, jnp.int32)],
                           plsc.cumsum(acc), mask=last_lane)
        return carry

    lax.fori_loop(0, _BPW, body, 0)

    pltpu.sync_copy(out_v, out_hbm.at[wid])


def kernel(inputs_0, inputs_1, factor_0, factor_1):
    idx0 = inputs_0.reshape(_NW, _NGRP, _L)
    idx1 = inputs_1.reshape(_NW, _NGRP, _L)
    out = _sc_dot_gather(idx0, idx1, factor_0.T, factor_1.T)
    return out.reshape(_B)
